# baseline (speedup 1.0000x reference)
import functools

import jax
import jax.numpy as jnp
from jax import lax
from jax.experimental import pallas as pl
from jax.experimental.pallas import tpu as pltpu
from jax.experimental.pallas import tpu_sc as plsc

N_NODES = 100000
N_EDGES = 3200000
NUM_REL = 16
D_IN = 7
D_HID = 16
D_OUT = 2
D_OUT_PAD = 8

NC, NS = 2, 16
NW = NC * NS
EB = 128
EPAD = 3276800
RTOT = EPAD // EB
ROWS_PER_TILE = RTOT // NW
NPAD = N_NODES + 96
RPT_OUT = NPAD // NS

BN = 2000
GRID_N = N_NODES // BN

ZROWS = 512
Z1DL = 8192


def _idx_body(src_ref, et_ref, idx_ref):
    idx_ref[...] = et_ref[...] * N_NODES + src_ref[...]


def _tc_idx(srcp, etp):
    blk = pl.BlockSpec((512, EB), lambda i: (i, 0))
    return pl.pallas_call(
        _idx_body,
        grid=(RTOT // 512,),
        in_specs=[blk, blk],
        out_specs=blk,
        out_shape=jax.ShapeDtypeStruct((RTOT, EB), jnp.int32),
    )(srcp, etp)


def _prep_body(x_ref, w_ref, y_ref):
    y_ref[...] = jnp.dot(x_ref[...], w_ref[0],
                         preferred_element_type=jnp.float32)


def _tc_prep(x, w1):
    return pl.pallas_call(
        _prep_body,
        grid=(GRID_N, NUM_REL),
        in_specs=[
            pl.BlockSpec((BN, D_IN), lambda i, r: (i, 0)),
            pl.BlockSpec((1, D_IN, D_HID), lambda i, r: (r, 0, 0)),
        ],
        out_specs=pl.BlockSpec((BN, D_HID), lambda i, r: (r * GRID_N + i, 0)),
        out_shape=jax.ShapeDtypeStruct((NUM_REL * N_NODES, D_HID),
                                       jnp.float32),
    )(x, w1)


def _make_sc_pass(width, kb, async_scatter):
    mesh = plsc.VectorSubcoreMesh(core_axis_name="c", subcore_axis_name="s",
                                  num_cores=NC, num_subcores=NS)
    outer = ROWS_PER_TILE // kb

    scratch = [
        pltpu.VMEM((2 * kb, EB), jnp.int32),
        pltpu.VMEM((2 * kb, EB), jnp.int32),
        pltpu.VMEM((2 * kb * EB, width), jnp.float32),
        pltpu.VMEM((ZROWS, width), jnp.float32),
        pltpu.VMEM_SHARED((NPAD, width), jnp.float32),
        pltpu.SemaphoreType.DMA,
        pltpu.SemaphoreType.DMA,
    ]
    if async_scatter:
        scratch += [pltpu.SemaphoreType.DMA, pltpu.SemaphoreType.DMA]
    out_type = [jax.ShapeDtypeStruct((NC, NPAD, width), jnp.float32)]

    @functools.partial(
        pl.kernel, out_type=out_type, mesh=mesh, scratch_types=scratch,
        compiler_params=pltpu.CompilerParams(use_tc_tiling_on_sc=False))
    def sc_pass(idx_hbm, dst_hbm, tab_hbm, zrow_hbm, *refs):
        if async_scatter:
            (sums_hbm, idx_v, dst_v, rows_v, zv, acc_sh,
             gsem0, gsem1, ssem0, ssem1) = refs
            ssem = (ssem0, ssem1)
        else:
            (sums_hbm, idx_v, dst_v, rows_v, zv, acc_sh,
             gsem0, gsem1) = refs
        gsem = (gsem0, gsem1)
        c = lax.axis_index("c")
        s = lax.axis_index("s")
        wid = c * NS + s

        pltpu.sync_copy(zrow_hbm, zv)
        zbase = s * RPT_OUT
        nfull = RPT_OUT // ZROWS
        for k in range(nfull):
            pltpu.sync_copy(zv, acc_sh.at[pl.ds(zbase + k * ZROWS, ZROWS)])
        rem = RPT_OUT - nfull * ZROWS
        pltpu.sync_copy(zv.at[pl.ds(0, rem)],
                        acc_sh.at[pl.ds(zbase + nfull * ZROWS, rem)])

        plsc.subcore_barrier()

        def _load_fire(b, chunk):
            r0 = wid * ROWS_PER_TILE + chunk * kb
            pltpu.sync_copy(idx_hbm.at[pl.ds(r0, kb)],
                            idx_v.at[pl.ds(b * kb, kb)])
            pltpu.sync_copy(dst_hbm.at[pl.ds(r0, kb)],
                            dst_v.at[pl.ds(b * kb, kb)])
            for j in range(kb):
                pltpu.async_copy(tab_hbm.at[idx_v.at[b * kb + j]],
                                 rows_v.at[pl.ds((b * kb + j) * EB, EB)],
                                 gsem[b])

        def _wait_g(b):
            for j in range(kb):
                pltpu.make_async_copy(
                    tab_hbm.at[idx_v.at[b * kb + j]],
                    rows_v.at[pl.ds((b * kb + j) * EB, EB)],
                    gsem[b]).wait()

        if async_scatter:
            def _fire_s(b):
                for j in range(kb):
                    pltpu.async_copy(
                        rows_v.at[pl.ds((b * kb + j) * EB, EB)],
                        acc_sh.at[dst_v.at[b * kb + j]], ssem[b], add=True)

            def _wait_s(b):
                for j in range(kb):
                    pltpu.make_async_copy(
                        tab_hbm.at[idx_v.at[b * kb + j]],
                        rows_v.at[pl.ds((b * kb + j) * EB, EB)],
                        ssem[b]).wait()

            _load_fire(0, 0)
            _load_fire(1, 1)

            def _pipe(k, carry):
                _wait_g(0)
                _fire_s(0)
                _wait_s(0)
                _load_fire(0, 2 * k + 2)
                _wait_g(1)
                _fire_s(1)
                _wait_s(1)
                _load_fire(1, 2 * k + 3)
                return carry
            lax.fori_loop(0, outer // 2 - 1, _pipe, 0)

            _wait_g(0)
            _fire_s(0)
            _wait_s(0)
            _wait_g(1)
            _fire_s(1)
            _wait_s(1)
        else:
            def _scat_sync(b):
                for j in range(kb):
                    pltpu.sync_copy(rows_v.at[pl.ds((b * kb + j) * EB, EB)],
                                    acc_sh.at[dst_v.at[b * kb + j]],
                                    add=True)

            _load_fire(0, 0)

            def _pipe(k, carry):
                _wait_g(0)
                _load_fire(1, 2 * k + 1)
                _scat_sync(0)
                _wait_g(1)
                _load_fire(0, 2 * k + 2)
                _scat_sync(1)
                return carry
            lax.fori_loop(0, outer // 2 - 1, _pipe, 0)

            _wait_g(0)
            _load_fire(1, outer - 1)
            _scat_sync(0)
            _wait_g(1)
            _scat_sync(1)

        plsc.subcore_barrier()

        ob = s * RPT_OUT
        pltpu.sync_copy(acc_sh.at[pl.ds(ob, RPT_OUT)],
                        sums_hbm.at[c, pl.ds(ob, RPT_OUT)])

    return sc_pass


_sc_pass1 = _make_sc_pass(D_HID, 4, async_scatter=False)
_sc_pass2 = _make_sc_pass(D_OUT_PAD, 4, async_scatter=True)


def _make_sc_cnt(kb):
    mesh = plsc.VectorSubcoreMesh(core_axis_name="c", subcore_axis_name="s",
                                  num_cores=NC, num_subcores=NS)
    outer = ROWS_PER_TILE // kb
    scratch = [
        pltpu.VMEM((kb, EB), jnp.int32),
        pltpu.VMEM((EB,), jnp.float32),
        pltpu.VMEM((Z1DL,), jnp.float32),
        pltpu.VMEM_SHARED((NPAD,), jnp.float32),
    ]
    out_type = [jax.ShapeDtypeStruct((NPAD,), jnp.float32),
                jax.ShapeDtypeStruct((NPAD,), jnp.float32)]

    @functools.partial(
        pl.kernel, out_type=out_type, mesh=mesh, scratch_types=scratch,
        compiler_params=pltpu.CompilerParams(use_tc_tiling_on_sc=False))
    def sc_cnt(dst_hbm, z1d_hbm, cnts0_hbm, cnts1_hbm, dst_v, ones_v, z1_v,
               cnt_sh):
        c = lax.axis_index("c")
        s = lax.axis_index("s")
        wid = c * NS + s

        pltpu.sync_copy(z1d_hbm, z1_v)

        @pl.when(s == 0)
        def _zero_cnt():
            nf1 = NPAD // Z1DL
            for k in range(nf1):
                pltpu.sync_copy(z1_v, cnt_sh.at[pl.ds(k * Z1DL, Z1DL)])
            r1 = NPAD - nf1 * Z1DL
            pltpu.sync_copy(z1_v.at[pl.ds(0, r1)],
                            cnt_sh.at[pl.ds(nf1 * Z1DL, r1)])

        def _init_ones(i, carry):
            ones_v[pl.ds(i * 16, 16)] = jnp.full((16,), 1.0, jnp.float32)
            return carry
        lax.fori_loop(0, EB // 16, _init_ones, 0)

        plsc.subcore_barrier()

        def _step(jo, carry):
            r0 = wid * ROWS_PER_TILE + jo * kb
            pltpu.sync_copy(dst_hbm.at[pl.ds(r0, kb)], dst_v)
            for j in range(kb):
                pltpu.sync_copy(ones_v, cnt_sh.at[dst_v.at[j]], add=True)
            return carry
        lax.fori_loop(0, outer, _step, 0)

        plsc.subcore_barrier()

        @pl.when((s == 0) & (c == 0))
        def _cnt_out0():
            pltpu.sync_copy(cnt_sh, cnts0_hbm)

        @pl.when((s == 0) & (c == 1))
        def _cnt_out1():
            pltpu.sync_copy(cnt_sh, cnts1_hbm)

    return sc_cnt


_sc_cnt = _make_sc_cnt(8)


def _mid_body(sums_ref, cnts_ref, x_ref, root1_ref, b1_ref, w2_ref,
              root2_ref, b2_ref, y2_ref, xr2_ref):
    agg = sums_ref[0] + sums_ref[1]
    cnt = cnts_ref[0, :, 0] + cnts_ref[1, :, 0]
    inv = 1.0 / jnp.maximum(cnt, 1.0)
    h = agg * inv[:, None] + jnp.dot(
        x_ref[...], root1_ref[...], preferred_element_type=jnp.float32)
    h = jnp.maximum(h + b1_ref[...], 0.0)
    y2_ref[...] = jnp.dot(h, w2_ref[0], preferred_element_type=jnp.float32)
    xr2_ref[...] = jnp.dot(h, root2_ref[...],
                           preferred_element_type=jnp.float32) + b2_ref[...]


def _tc_mid(sums1, cnts, x, root1, b1r, w2p, root2p, b2r):
    return pl.pallas_call(
        _mid_body,
        grid=(GRID_N, NUM_REL),
        in_specs=[
            pl.BlockSpec((NC, BN, D_HID), lambda i, r: (0, i, 0)),
            pl.BlockSpec((NC, BN, 1), lambda i, r: (0, i, 0)),
            pl.BlockSpec((BN, D_IN), lambda i, r: (i, 0)),
            pl.BlockSpec((D_IN, D_HID), lambda i, r: (0, 0)),
            pl.BlockSpec((1, D_HID), lambda i, r: (0, 0)),
            pl.BlockSpec((1, D_HID, D_OUT_PAD), lambda i, r: (r, 0, 0)),
            pl.BlockSpec((D_HID, D_OUT_PAD), lambda i, r: (0, 0)),
            pl.BlockSpec((1, D_OUT_PAD), lambda i, r: (0, 0)),
        ],
        out_specs=[
            pl.BlockSpec((BN, D_OUT_PAD), lambda i, r: (r * GRID_N + i, 0)),
            pl.BlockSpec((BN, D_OUT_PAD), lambda i, r: (i, 0)),
        ],
        out_shape=[
            jax.ShapeDtypeStruct((NUM_REL * N_NODES, D_OUT_PAD),
                                 jnp.float32),
            jax.ShapeDtypeStruct((N_NODES, D_OUT_PAD), jnp.float32),
        ],
    )(sums1, cnts, x, root1, b1r, w2p, root2p, b2r)


def _final_body(sums_ref, cnts_ref, xr2_ref, out_ref):
    agg = sums_ref[0] + sums_ref[1]
    cnt = cnts_ref[0, :, 0] + cnts_ref[1, :, 0]
    inv = 1.0 / jnp.maximum(cnt, 1.0)
    z = agg * inv[:, None] + xr2_ref[...]
    lane = lax.broadcasted_iota(jnp.int32, z.shape, 1)
    zm = jnp.where(lane < D_OUT, z, -jnp.inf)
    m = jnp.max(zm, axis=1, keepdims=True)
    lse = m + jnp.log(jnp.sum(jnp.where(lane < D_OUT, jnp.exp(z - m), 0.0),
                              axis=1, keepdims=True))
    out_ref[...] = (z - lse)[:, :D_OUT]


def _tc_final(sums2, cnts, xr2):
    return pl.pallas_call(
        _final_body,
        grid=(GRID_N,),
        in_specs=[
            pl.BlockSpec((NC, BN, D_OUT_PAD), lambda i: (0, i, 0)),
            pl.BlockSpec((NC, BN, 1), lambda i: (0, i, 0)),
            pl.BlockSpec((BN, D_OUT_PAD), lambda i: (i, 0)),
        ],
        out_specs=pl.BlockSpec((BN, D_OUT), lambda i: (i, 0)),
        out_shape=jax.ShapeDtypeStruct((N_NODES, D_OUT), jnp.float32),
    )(sums2, cnts, xr2)


def kernel(x, edge_index, edge_type, W1, root1, b1, W2, root2, b2):
    pad = EPAD - N_EDGES
    srcp = jnp.pad(edge_index[0], (0, pad)).reshape(RTOT, EB)
    etp = jnp.pad(edge_type, (0, pad)).reshape(RTOT, EB)
    dstp = jnp.pad(edge_index[1], (0, pad),
                   constant_values=N_NODES).reshape(RTOT, EB)

    idxp = _tc_idx(srcp, etp)
    y1 = _tc_prep(x, W1)

    zrow16 = jnp.zeros((ZROWS, D_HID), jnp.float32)
    z1d = jnp.zeros((Z1DL,), jnp.float32)
    cnts0, cnts1 = _sc_cnt(dstp, z1d)
    cnts3 = jnp.stack([cnts0, cnts1]).reshape(NC, NPAD, 1)
    (sums1,) = _sc_pass1(idxp, dstp, y1, zrow16)

    w2p = jnp.pad(W2, ((0, 0), (0, 0), (0, D_OUT_PAD - D_OUT)))
    root2p = jnp.pad(root2, ((0, 0), (0, D_OUT_PAD - D_OUT)))
    b2r = jnp.pad(b2, (0, D_OUT_PAD - D_OUT)).reshape(1, D_OUT_PAD)
    y2, xr2 = _tc_mid(sums1, cnts3, x, root1, b1.reshape(1, D_HID),
                      w2p, root2p, b2r)

    zrow8 = jnp.zeros((ZROWS, D_OUT_PAD), jnp.float32)
    (sums2,) = _sc_pass2(idxp, dstp, y2, zrow8)
    return _tc_final(sums2, cnts3, xr2)


# speedup vs baseline: 2.4008x; 2.4008x over previous
import functools

import jax
import jax.numpy as jnp
from jax import lax
from jax.experimental import pallas as pl
from jax.experimental.pallas import tpu as pltpu
from jax.experimental.pallas import tpu_sc as plsc

N_NODES = 100000
N_EDGES = 3200000
NUM_REL = 16
D_IN = 7
D_HID = 16
D_OUT = 2
D_OUT_PAD = 8

NC, NS = 2, 16
NW = NC * NS
EB = 128
EPAD = 3276800
RTOT = EPAD // EB
ROWS_PER_TILE = RTOT // NW
NPAD = N_NODES + 96
RPT_OUT = NPAD // NS

BN = 2000
GRID_N = N_NODES // BN

ZROWS = 512
Z1DL = 8192


def _idx_body(src_ref, et_ref, idx_ref):
    idx_ref[...] = et_ref[...] * N_NODES + src_ref[...]


def _tc_idx(srcp, etp):
    blk = pl.BlockSpec((512, EB), lambda i: (i, 0))
    return pl.pallas_call(
        _idx_body,
        grid=(RTOT // 512,),
        in_specs=[blk, blk],
        out_specs=blk,
        out_shape=jax.ShapeDtypeStruct((RTOT, EB), jnp.int32),
    )(srcp, etp)


NG8 = N_NODES // 8
NG16 = N_NODES // 16


def _prep_body(xg_ref, w_ref, y_ref):
    xg = xg_ref[...]
    ya = jnp.dot(xg, w_ref[0], preferred_element_type=jnp.float32)
    yb = jnp.dot(xg, w_ref[1], preferred_element_type=jnp.float32)
    y_ref[...] = jnp.concatenate([ya, yb], axis=0)


def _tc_prep(xg, wbig1):
    return pl.pallas_call(
        _prep_body,
        grid=(NUM_REL // 2,),
        in_specs=[
            pl.BlockSpec((NG8, 8 * D_IN), lambda p: (0, 0)),
            pl.BlockSpec((2, 8 * D_IN, 128), lambda p: (p, 0, 0)),
        ],
        out_specs=pl.BlockSpec((2 * NG8, 128), lambda p: (p, 0)),
        out_shape=jax.ShapeDtypeStruct((NUM_REL * NG8, 128), jnp.float32),
    )(xg, wbig1)


def _make_sc_pass(width, kb, async_scatter):
    mesh = plsc.VectorSubcoreMesh(core_axis_name="c", subcore_axis_name="s",
                                  num_cores=NC, num_subcores=NS)
    outer = ROWS_PER_TILE // kb

    scratch = [
        pltpu.VMEM((2 * kb, EB), jnp.int32),
        pltpu.VMEM((2 * kb, EB), jnp.int32),
        pltpu.VMEM((2 * kb * EB, width), jnp.float32),
        pltpu.VMEM((ZROWS, width), jnp.float32),
        pltpu.VMEM_SHARED((NPAD, width), jnp.float32),
        pltpu.SemaphoreType.DMA,
        pltpu.SemaphoreType.DMA,
    ]
    if async_scatter:
        scratch += [pltpu.SemaphoreType.DMA, pltpu.SemaphoreType.DMA]
    out_type = [jax.ShapeDtypeStruct((NC, NPAD, width), jnp.float32)]

    @functools.partial(
        pl.kernel, out_type=out_type, mesh=mesh, scratch_types=scratch,
        compiler_params=pltpu.CompilerParams(use_tc_tiling_on_sc=False))
    def sc_pass(idx_hbm, dst_hbm, tab_hbm, zrow_hbm, *refs):
        if async_scatter:
            (sums_hbm, idx_v, dst_v, rows_v, zv, acc_sh,
             gsem0, gsem1, ssem0, ssem1) = refs
            ssem = (ssem0, ssem1)
        else:
            (sums_hbm, idx_v, dst_v, rows_v, zv, acc_sh,
             gsem0, gsem1) = refs
        gsem = (gsem0, gsem1)
        c = lax.axis_index("c")
        s = lax.axis_index("s")
        wid = c * NS + s

        pltpu.sync_copy(zrow_hbm, zv)
        zbase = s * RPT_OUT
        nfull = RPT_OUT // ZROWS
        for k in range(nfull):
            pltpu.sync_copy(zv, acc_sh.at[pl.ds(zbase + k * ZROWS, ZROWS)])
        rem = RPT_OUT - nfull * ZROWS
        pltpu.sync_copy(zv.at[pl.ds(0, rem)],
                        acc_sh.at[pl.ds(zbase + nfull * ZROWS, rem)])

        plsc.subcore_barrier()

        def _load_fire(b, chunk):
            r0 = wid * ROWS_PER_TILE + chunk * kb
            pltpu.sync_copy(idx_hbm.at[pl.ds(r0, kb)],
                            idx_v.at[pl.ds(b * kb, kb)])
            pltpu.sync_copy(dst_hbm.at[pl.ds(r0, kb)],
                            dst_v.at[pl.ds(b * kb, kb)])
            for j in range(kb):
                pltpu.async_copy(tab_hbm.at[idx_v.at[b * kb + j]],
                                 rows_v.at[pl.ds((b * kb + j) * EB, EB)],
                                 gsem[b])

        def _wait_g(b):
            for j in range(kb):
                pltpu.make_async_copy(
                    tab_hbm.at[idx_v.at[b * kb + j]],
                    rows_v.at[pl.ds((b * kb + j) * EB, EB)],
                    gsem[b]).wait()

        if async_scatter:
            def _fire_s(b):
                for j in range(kb):
                    pltpu.async_copy(
                        rows_v.at[pl.ds((b * kb + j) * EB, EB)],
                        acc_sh.at[dst_v.at[b * kb + j]], ssem[b], add=True)

            def _wait_s(b):
                for j in range(kb):
                    pltpu.make_async_copy(
                        tab_hbm.at[idx_v.at[b * kb + j]],
                        rows_v.at[pl.ds((b * kb + j) * EB, EB)],
                        ssem[b]).wait()

            _load_fire(0, 0)
            _load_fire(1, 1)

            def _pipe(k, carry):
                _wait_g(0)
                _fire_s(0)
                _wait_s(0)
                _load_fire(0, 2 * k + 2)
                _wait_g(1)
                _fire_s(1)
                _wait_s(1)
                _load_fire(1, 2 * k + 3)
                return carry
            lax.fori_loop(0, outer // 2 - 1, _pipe, 0)

            _wait_g(0)
            _fire_s(0)
            _wait_s(0)
            _wait_g(1)
            _fire_s(1)
            _wait_s(1)
        else:
            def _scat_sync(b):
                for j in range(kb):
                    pltpu.sync_copy(rows_v.at[pl.ds((b * kb + j) * EB, EB)],
                                    acc_sh.at[dst_v.at[b * kb + j]],
                                    add=True)

            _load_fire(0, 0)

            def _pipe(k, carry):
                _wait_g(0)
                _load_fire(1, 2 * k + 1)
                _scat_sync(0)
                _wait_g(1)
                _load_fire(0, 2 * k + 2)
                _scat_sync(1)
                return carry
            lax.fori_loop(0, outer // 2 - 1, _pipe, 0)

            _wait_g(0)
            _load_fire(1, outer - 1)
            _scat_sync(0)
            _wait_g(1)
            _scat_sync(1)

        plsc.subcore_barrier()

        ob = s * RPT_OUT
        pltpu.sync_copy(acc_sh.at[pl.ds(ob, RPT_OUT)],
                        sums_hbm.at[c, pl.ds(ob, RPT_OUT)])

    return sc_pass


_sc_pass1 = _make_sc_pass(D_HID, 4, async_scatter=True)
_sc_pass2 = _make_sc_pass(D_OUT_PAD, 4, async_scatter=True)


def _make_sc_cnt(kb):
    mesh = plsc.VectorSubcoreMesh(core_axis_name="c", subcore_axis_name="s",
                                  num_cores=NC, num_subcores=NS)
    outer = ROWS_PER_TILE // kb
    scratch = [
        pltpu.VMEM((kb, EB), jnp.int32),
        pltpu.VMEM((EB,), jnp.float32),
        pltpu.VMEM((Z1DL,), jnp.float32),
        pltpu.VMEM_SHARED((NPAD,), jnp.float32),
    ]
    out_type = [jax.ShapeDtypeStruct((NPAD,), jnp.float32),
                jax.ShapeDtypeStruct((NPAD,), jnp.float32)]

    @functools.partial(
        pl.kernel, out_type=out_type, mesh=mesh, scratch_types=scratch,
        compiler_params=pltpu.CompilerParams(use_tc_tiling_on_sc=False))
    def sc_cnt(dst_hbm, z1d_hbm, cnts0_hbm, cnts1_hbm, dst_v, ones_v, z1_v,
               cnt_sh):
        c = lax.axis_index("c")
        s = lax.axis_index("s")
        wid = c * NS + s

        pltpu.sync_copy(z1d_hbm, z1_v)

        @pl.when(s == 0)
        def _zero_cnt():
            nf1 = NPAD // Z1DL
            for k in range(nf1):
                pltpu.sync_copy(z1_v, cnt_sh.at[pl.ds(k * Z1DL, Z1DL)])
            r1 = NPAD - nf1 * Z1DL
            pltpu.sync_copy(z1_v.at[pl.ds(0, r1)],
                            cnt_sh.at[pl.ds(nf1 * Z1DL, r1)])

        def _init_ones(i, carry):
            ones_v[pl.ds(i * 16, 16)] = jnp.full((16,), 1.0, jnp.float32)
            return carry
        lax.fori_loop(0, EB // 16, _init_ones, 0)

        plsc.subcore_barrier()

        def _step(jo, carry):
            r0 = wid * ROWS_PER_TILE + jo * kb
            pltpu.sync_copy(dst_hbm.at[pl.ds(r0, kb)], dst_v)
            for j in range(kb):
                pltpu.sync_copy(ones_v, cnt_sh.at[dst_v.at[j]], add=True)
            return carry
        lax.fori_loop(0, outer, _step, 0)

        plsc.subcore_barrier()

        @pl.when((s == 0) & (c == 0))
        def _cnt_out0():
            pltpu.sync_copy(cnt_sh, cnts0_hbm)

        @pl.when((s == 0) & (c == 1))
        def _cnt_out1():
            pltpu.sync_copy(cnt_sh, cnts1_hbm)

    return sc_cnt


_sc_cnt = _make_sc_cnt(8)


def _mid_a_body(sums_ref, cnts_ref, x_ref, root1_ref, b1_ref,
                root2_ref, b2_ref, h_ref, xr2_ref):
    agg = sums_ref[0] + sums_ref[1]
    cnt = cnts_ref[0, :, 0] + cnts_ref[1, :, 0]
    inv = 1.0 / jnp.maximum(cnt, 1.0)
    h = agg * inv[:, None] + jnp.dot(
        x_ref[...], root1_ref[...], preferred_element_type=jnp.float32)
    h = jnp.maximum(h + b1_ref[...], 0.0)
    h_ref[...] = h
    xr2_ref[...] = jnp.dot(h, root2_ref[...],
                           preferred_element_type=jnp.float32) + b2_ref[...]


def _tc_mid_a(sums1, cnts, x, root1, b1r, root2p, b2r):
    return pl.pallas_call(
        _mid_a_body,
        grid=(GRID_N,),
        in_specs=[
            pl.BlockSpec((NC, BN, D_HID), lambda i: (0, i, 0)),
            pl.BlockSpec((NC, BN, 1), lambda i: (0, i, 0)),
            pl.BlockSpec((BN, D_IN), lambda i: (i, 0)),
            pl.BlockSpec((D_IN, D_HID), lambda i: (0, 0)),
            pl.BlockSpec((1, D_HID), lambda i: (0, 0)),
            pl.BlockSpec((D_HID, D_OUT_PAD), lambda i: (0, 0)),
            pl.BlockSpec((1, D_OUT_PAD), lambda i: (0, 0)),
        ],
        out_specs=[
            pl.BlockSpec((BN, D_HID), lambda i: (i, 0)),
            pl.BlockSpec((BN, D_OUT_PAD), lambda i: (i, 0)),
        ],
        out_shape=[
            jax.ShapeDtypeStruct((N_NODES, D_HID), jnp.float32),
            jax.ShapeDtypeStruct((N_NODES, D_OUT_PAD), jnp.float32),
        ],
    )(sums1, cnts, x, root1, b1r, root2p, b2r)


def _mid_b_body(hg_ref, w_ref, y_ref):
    hg = hg_ref[...]
    ys = [jnp.dot(hg, w_ref[k], preferred_element_type=jnp.float32)
          for k in range(4)]
    y_ref[...] = jnp.concatenate(ys, axis=0)


def _tc_mid_b(hg, wbig2):
    return pl.pallas_call(
        _mid_b_body,
        grid=(NUM_REL // 4,),
        in_specs=[
            pl.BlockSpec((NG16, 16 * D_HID), lambda q: (0, 0)),
            pl.BlockSpec((4, 16 * D_HID, 128), lambda q: (q, 0, 0)),
        ],
        out_specs=pl.BlockSpec((4 * NG16, 128), lambda q: (q, 0)),
        out_shape=jax.ShapeDtypeStruct((NUM_REL * NG16, 128), jnp.float32),
    )(hg, wbig2)


def _final_body(sums_ref, cnts_ref, xr2_ref, out_ref):
    agg = sums_ref[0] + sums_ref[1]
    cnt = cnts_ref[0, :, 0] + cnts_ref[1, :, 0]
    inv = 1.0 / jnp.maximum(cnt, 1.0)
    z = agg * inv[:, None] + xr2_ref[...]
    lane = lax.broadcasted_iota(jnp.int32, z.shape, 1)
    zm = jnp.where(lane < D_OUT, z, -jnp.inf)
    m = jnp.max(zm, axis=1, keepdims=True)
    lse = m + jnp.log(jnp.sum(jnp.where(lane < D_OUT, jnp.exp(z - m), 0.0),
                              axis=1, keepdims=True))
    out_ref[...] = (z - lse)[:, :D_OUT]


def _tc_final(sums2, cnts, xr2):
    return pl.pallas_call(
        _final_body,
        grid=(GRID_N,),
        in_specs=[
            pl.BlockSpec((NC, BN, D_OUT_PAD), lambda i: (0, i, 0)),
            pl.BlockSpec((NC, BN, 1), lambda i: (0, i, 0)),
            pl.BlockSpec((BN, D_OUT_PAD), lambda i: (i, 0)),
        ],
        out_specs=pl.BlockSpec((BN, D_OUT), lambda i: (i, 0)),
        out_shape=jax.ShapeDtypeStruct((N_NODES, D_OUT), jnp.float32),
    )(sums2, cnts, xr2)


def kernel(x, edge_index, edge_type, W1, root1, b1, W2, root2, b2):
    pad = EPAD - N_EDGES
    srcp = jnp.pad(edge_index[0], (0, pad)).reshape(RTOT, EB)
    etp = jnp.pad(edge_type, (0, pad)).reshape(RTOT, EB)
    dstp = jnp.pad(edge_index[1], (0, pad),
                   constant_values=N_NODES).reshape(RTOT, EB)

    idxp = _tc_idx(srcp, etp)

    xg = x.reshape(NG8, 8 * D_IN)
    eye8 = jnp.eye(8, dtype=jnp.float32)
    wbig1 = jnp.einsum('jk,rfo->rjfko', eye8, W1).reshape(
        NUM_REL, 8 * D_IN, 8 * D_HID)
    y1p = _tc_prep(xg, wbig1)
    y1 = y1p.reshape(NUM_REL * N_NODES, D_HID)

    zrow16 = jnp.zeros((ZROWS, D_HID), jnp.float32)
    z1d = jnp.zeros((Z1DL,), jnp.float32)
    cnts0, cnts1 = _sc_cnt(dstp, z1d)
    cnts3 = jnp.stack([cnts0, cnts1]).reshape(NC, NPAD, 1)
    (sums1,) = _sc_pass1(idxp, dstp, y1, zrow16)

    w2p = jnp.pad(W2, ((0, 0), (0, 0), (0, D_OUT_PAD - D_OUT)))
    root2p = jnp.pad(root2, ((0, 0), (0, D_OUT_PAD - D_OUT)))
    b2r = jnp.pad(b2, (0, D_OUT_PAD - D_OUT)).reshape(1, D_OUT_PAD)
    eye16 = jnp.eye(16, dtype=jnp.float32)
    wbig2 = jnp.einsum('jk,rfo->rjfko', eye16, w2p).reshape(
        NUM_REL, 16 * D_HID, 16 * D_OUT_PAD)
    h, xr2 = _tc_mid_a(sums1, cnts3, x, root1, b1.reshape(1, D_HID),
                       root2p, b2r)
    hg = h.reshape(NG16, 16 * D_HID)
    y2p = _tc_mid_b(hg, wbig2)
    y2 = y2p.reshape(NUM_REL * N_NODES, D_OUT_PAD)

    zrow8 = jnp.zeros((ZROWS, D_OUT_PAD), jnp.float32)
    (sums2,) = _sc_pass2(idxp, dstp, y2, zrow8)
    return _tc_final(sums2, cnts3, xr2)


# speedup vs baseline: 2.4442x; 1.0181x over previous
import functools

import jax
import jax.numpy as jnp
from jax import lax
from jax.experimental import pallas as pl
from jax.experimental.pallas import tpu as pltpu
from jax.experimental.pallas import tpu_sc as plsc

N_NODES = 100000
N_EDGES = 3200000
NUM_REL = 16
D_IN = 7
D_HID = 16
D_OUT = 2
D_OUT_PAD = 8

NC, NS = 2, 16
NW = NC * NS
EB = 128
EPAD = 3276800
RTOT = EPAD // EB
ROWS_PER_TILE = RTOT // NW
NPAD = N_NODES + 96
RPT_OUT = NPAD // NS

BN = 2000
GRID_N = N_NODES // BN

ZROWS = 512
Z1DL = 8192


def _idx_body(src_ref, et_ref, idx_ref):
    idx_ref[...] = et_ref[...] * N_NODES + src_ref[...]


def _tc_idx(srcp, etp):
    blk = pl.BlockSpec((512, EB), lambda i: (i, 0))
    return pl.pallas_call(
        _idx_body,
        grid=(RTOT // 512,),
        in_specs=[blk, blk],
        out_specs=blk,
        out_shape=jax.ShapeDtypeStruct((RTOT, EB), jnp.int32),
    )(srcp, etp)


NG8 = N_NODES // 8
NG16 = N_NODES // 16


def _prep_body(xg_ref, w_ref, y_ref):
    xg = xg_ref[...]
    ya = jnp.dot(xg, w_ref[0], preferred_element_type=jnp.float32)
    yb = jnp.dot(xg, w_ref[1], preferred_element_type=jnp.float32)
    y_ref[...] = jnp.concatenate([ya, yb], axis=0)


def _tc_prep(xg, wbig1):
    return pl.pallas_call(
        _prep_body,
        grid=(NUM_REL // 2,),
        in_specs=[
            pl.BlockSpec((NG8, 8 * D_IN), lambda p: (0, 0)),
            pl.BlockSpec((2, 8 * D_IN, 128), lambda p: (p, 0, 0)),
        ],
        out_specs=pl.BlockSpec((2 * NG8, 128), lambda p: (p, 0)),
        out_shape=jax.ShapeDtypeStruct((NUM_REL * NG8, 128), jnp.float32),
    )(xg, wbig1)


def _make_sc_pass(width, kb, async_scatter):
    mesh = plsc.VectorSubcoreMesh(core_axis_name="c", subcore_axis_name="s",
                                  num_cores=NC, num_subcores=NS)
    outer = ROWS_PER_TILE // kb

    scratch = [
        pltpu.VMEM((2 * kb, EB), jnp.int32),
        pltpu.VMEM((2 * kb, EB), jnp.int32),
        pltpu.VMEM((2 * kb * EB, width), jnp.float32),
        pltpu.VMEM((ZROWS, width), jnp.float32),
        pltpu.VMEM_SHARED((NPAD, width), jnp.float32),
        pltpu.SemaphoreType.DMA,
        pltpu.SemaphoreType.DMA,
    ]
    if async_scatter:
        scratch += [pltpu.SemaphoreType.DMA, pltpu.SemaphoreType.DMA]
    out_type = [jax.ShapeDtypeStruct((NC, NPAD, width), jnp.float32)]

    @functools.partial(
        pl.kernel, out_type=out_type, mesh=mesh, scratch_types=scratch,
        compiler_params=pltpu.CompilerParams(use_tc_tiling_on_sc=False))
    def sc_pass(idx_hbm, dst_hbm, tab_hbm, zrow_hbm, *refs):
        if async_scatter:
            (sums_hbm, idx_v, dst_v, rows_v, zv, acc_sh,
             gsem0, gsem1, ssem0, ssem1) = refs
            ssem = (ssem0, ssem1)
        else:
            (sums_hbm, idx_v, dst_v, rows_v, zv, acc_sh,
             gsem0, gsem1) = refs
        gsem = (gsem0, gsem1)
        c = lax.axis_index("c")
        s = lax.axis_index("s")
        wid = c * NS + s

        pltpu.sync_copy(zrow_hbm, zv)
        zbase = s * RPT_OUT
        nfull = RPT_OUT // ZROWS
        for k in range(nfull):
            pltpu.sync_copy(zv, acc_sh.at[pl.ds(zbase + k * ZROWS, ZROWS)])
        rem = RPT_OUT - nfull * ZROWS
        pltpu.sync_copy(zv.at[pl.ds(0, rem)],
                        acc_sh.at[pl.ds(zbase + nfull * ZROWS, rem)])

        plsc.subcore_barrier()

        def _load_fire(b, chunk):
            r0 = wid * ROWS_PER_TILE + chunk * kb
            pltpu.sync_copy(idx_hbm.at[pl.ds(r0, kb)],
                            idx_v.at[pl.ds(b * kb, kb)])
            pltpu.sync_copy(dst_hbm.at[pl.ds(r0, kb)],
                            dst_v.at[pl.ds(b * kb, kb)])
            for j in range(kb):
                pltpu.async_copy(tab_hbm.at[idx_v.at[b * kb + j]],
                                 rows_v.at[pl.ds((b * kb + j) * EB, EB)],
                                 gsem[b])

        def _drain(sem, b):
            nwait = (kb * EB) // ZROWS
            for t in range(nwait):
                pltpu.make_async_copy(
                    zrow_hbm,
                    rows_v.at[pl.ds(b * kb * EB + t * ZROWS, ZROWS)],
                    sem).wait()

        def _wait_g(b):
            _drain(gsem[b], b)

        if async_scatter:
            def _fire_s(b):
                for j in range(kb):
                    pltpu.async_copy(
                        rows_v.at[pl.ds((b * kb + j) * EB, EB)],
                        acc_sh.at[dst_v.at[b * kb + j]], ssem[b], add=True)

            def _wait_s(b):
                _drain(ssem[b], b)

            _load_fire(0, 0)
            _load_fire(1, 1)

            def _pipe(k, carry):
                _wait_g(0)
                _fire_s(0)
                _wait_s(0)
                _load_fire(0, 2 * k + 2)
                _wait_g(1)
                _fire_s(1)
                _wait_s(1)
                _load_fire(1, 2 * k + 3)
                return carry
            lax.fori_loop(0, outer // 2 - 1, _pipe, 0)

            _wait_g(0)
            _fire_s(0)
            _wait_s(0)
            _wait_g(1)
            _fire_s(1)
            _wait_s(1)
        else:
            def _scat_sync(b):
                for j in range(kb):
                    pltpu.sync_copy(rows_v.at[pl.ds((b * kb + j) * EB, EB)],
                                    acc_sh.at[dst_v.at[b * kb + j]],
                                    add=True)

            _load_fire(0, 0)

            def _pipe(k, carry):
                _wait_g(0)
                _load_fire(1, 2 * k + 1)
                _scat_sync(0)
                _wait_g(1)
                _load_fire(0, 2 * k + 2)
                _scat_sync(1)
                return carry
            lax.fori_loop(0, outer // 2 - 1, _pipe, 0)

            _wait_g(0)
            _load_fire(1, outer - 1)
            _scat_sync(0)
            _wait_g(1)
            _scat_sync(1)

        plsc.subcore_barrier()

        ob = s * RPT_OUT
        pltpu.sync_copy(acc_sh.at[pl.ds(ob, RPT_OUT)],
                        sums_hbm.at[c, pl.ds(ob, RPT_OUT)])

    return sc_pass


_sc_pass1 = _make_sc_pass(D_HID, 4, async_scatter=True)
_sc_pass2 = _make_sc_pass(D_OUT_PAD, 8, async_scatter=True)


def _make_sc_cnt(kb):
    mesh = plsc.VectorSubcoreMesh(core_axis_name="c", subcore_axis_name="s",
                                  num_cores=NC, num_subcores=NS)
    outer = ROWS_PER_TILE // kb
    scratch = [
        pltpu.VMEM((kb, EB), jnp.int32),
        pltpu.VMEM((EB,), jnp.float32),
        pltpu.VMEM((Z1DL,), jnp.float32),
        pltpu.VMEM_SHARED((NPAD,), jnp.float32),
    ]
    out_type = [jax.ShapeDtypeStruct((NPAD,), jnp.float32),
                jax.ShapeDtypeStruct((NPAD,), jnp.float32)]

    @functools.partial(
        pl.kernel, out_type=out_type, mesh=mesh, scratch_types=scratch,
        compiler_params=pltpu.CompilerParams(use_tc_tiling_on_sc=False))
    def sc_cnt(dst_hbm, z1d_hbm, cnts0_hbm, cnts1_hbm, dst_v, ones_v, z1_v,
               cnt_sh):
        c = lax.axis_index("c")
        s = lax.axis_index("s")
        wid = c * NS + s

        pltpu.sync_copy(z1d_hbm, z1_v)

        @pl.when(s == 0)
        def _zero_cnt():
            nf1 = NPAD // Z1DL
            for k in range(nf1):
                pltpu.sync_copy(z1_v, cnt_sh.at[pl.ds(k * Z1DL, Z1DL)])
            r1 = NPAD - nf1 * Z1DL
            pltpu.sync_copy(z1_v.at[pl.ds(0, r1)],
                            cnt_sh.at[pl.ds(nf1 * Z1DL, r1)])

        def _init_ones(i, carry):
            ones_v[pl.ds(i * 16, 16)] = jnp.full((16,), 1.0, jnp.float32)
            return carry
        lax.fori_loop(0, EB // 16, _init_ones, 0)

        plsc.subcore_barrier()

        def _step(jo, carry):
            r0 = wid * ROWS_PER_TILE + jo * kb
            pltpu.sync_copy(dst_hbm.at[pl.ds(r0, kb)], dst_v)
            for j in range(kb):
                pltpu.sync_copy(ones_v, cnt_sh.at[dst_v.at[j]], add=True)
            return carry
        lax.fori_loop(0, outer, _step, 0)

        plsc.subcore_barrier()

        @pl.when((s == 0) & (c == 0))
        def _cnt_out0():
            pltpu.sync_copy(cnt_sh, cnts0_hbm)

        @pl.when((s == 0) & (c == 1))
        def _cnt_out1():
            pltpu.sync_copy(cnt_sh, cnts1_hbm)

    return sc_cnt


_sc_cnt = _make_sc_cnt(8)


def _mid_a_body(sums_ref, cnts_ref, x_ref, root1_ref, b1_ref,
                root2_ref, b2_ref, h_ref, xr2_ref):
    agg = sums_ref[0] + sums_ref[1]
    cnt = cnts_ref[0, :, 0] + cnts_ref[1, :, 0]
    inv = 1.0 / jnp.maximum(cnt, 1.0)
    h = agg * inv[:, None] + jnp.dot(
        x_ref[...], root1_ref[...], preferred_element_type=jnp.float32)
    h = jnp.maximum(h + b1_ref[...], 0.0)
    h_ref[...] = h
    xr2_ref[...] = jnp.dot(h, root2_ref[...],
                           preferred_element_type=jnp.float32) + b2_ref[...]


def _tc_mid_a(sums1, cnts, x, root1, b1r, root2p, b2r):
    return pl.pallas_call(
        _mid_a_body,
        grid=(GRID_N,),
        in_specs=[
            pl.BlockSpec((NC, BN, D_HID), lambda i: (0, i, 0)),
            pl.BlockSpec((NC, BN, 1), lambda i: (0, i, 0)),
            pl.BlockSpec((BN, D_IN), lambda i: (i, 0)),
            pl.BlockSpec((D_IN, D_HID), lambda i: (0, 0)),
            pl.BlockSpec((1, D_HID), lambda i: (0, 0)),
            pl.BlockSpec((D_HID, D_OUT_PAD), lambda i: (0, 0)),
            pl.BlockSpec((1, D_OUT_PAD), lambda i: (0, 0)),
        ],
        out_specs=[
            pl.BlockSpec((BN, D_HID), lambda i: (i, 0)),
            pl.BlockSpec((BN, D_OUT_PAD), lambda i: (i, 0)),
        ],
        out_shape=[
            jax.ShapeDtypeStruct((N_NODES, D_HID), jnp.float32),
            jax.ShapeDtypeStruct((N_NODES, D_OUT_PAD), jnp.float32),
        ],
    )(sums1, cnts, x, root1, b1r, root2p, b2r)


def _mid_b_body(hg_ref, w_ref, y_ref):
    hg = hg_ref[...]
    ys = [jnp.dot(hg, w_ref[k], preferred_element_type=jnp.float32)
          for k in range(4)]
    y_ref[...] = jnp.concatenate(ys, axis=0)


def _tc_mid_b(hg, wbig2):
    return pl.pallas_call(
        _mid_b_body,
        grid=(NUM_REL // 4,),
        in_specs=[
            pl.BlockSpec((NG16, 16 * D_HID), lambda q: (0, 0)),
            pl.BlockSpec((4, 16 * D_HID, 128), lambda q: (q, 0, 0)),
        ],
        out_specs=pl.BlockSpec((4 * NG16, 128), lambda q: (q, 0)),
        out_shape=jax.ShapeDtypeStruct((NUM_REL * NG16, 128), jnp.float32),
    )(hg, wbig2)


def _final_body(sums_ref, cnts_ref, xr2_ref, out_ref):
    agg = sums_ref[0] + sums_ref[1]
    cnt = cnts_ref[0, :, 0] + cnts_ref[1, :, 0]
    inv = 1.0 / jnp.maximum(cnt, 1.0)
    z = agg * inv[:, None] + xr2_ref[...]
    lane = lax.broadcasted_iota(jnp.int32, z.shape, 1)
    zm = jnp.where(lane < D_OUT, z, -jnp.inf)
    m = jnp.max(zm, axis=1, keepdims=True)
    lse = m + jnp.log(jnp.sum(jnp.where(lane < D_OUT, jnp.exp(z - m), 0.0),
                              axis=1, keepdims=True))
    out_ref[...] = (z - lse)[:, :D_OUT]


def _tc_final(sums2, cnts, xr2):
    return pl.pallas_call(
        _final_body,
        grid=(GRID_N,),
        in_specs=[
            pl.BlockSpec((NC, BN, D_OUT_PAD), lambda i: (0, i, 0)),
            pl.BlockSpec((NC, BN, 1), lambda i: (0, i, 0)),
            pl.BlockSpec((BN, D_OUT_PAD), lambda i: (i, 0)),
        ],
        out_specs=pl.BlockSpec((BN, D_OUT), lambda i: (i, 0)),
        out_shape=jax.ShapeDtypeStruct((N_NODES, D_OUT), jnp.float32),
    )(sums2, cnts, xr2)


def kernel(x, edge_index, edge_type, W1, root1, b1, W2, root2, b2):
    pad = EPAD - N_EDGES
    srcp = jnp.pad(edge_index[0], (0, pad)).reshape(RTOT, EB)
    etp = jnp.pad(edge_type, (0, pad)).reshape(RTOT, EB)
    dstp = jnp.pad(edge_index[1], (0, pad),
                   constant_values=N_NODES).reshape(RTOT, EB)

    idxp = _tc_idx(srcp, etp)

    xg = x.reshape(NG8, 8 * D_IN)
    eye8 = jnp.eye(8, dtype=jnp.float32)
    wbig1 = jnp.einsum('jk,rfo->rjfko', eye8, W1).reshape(
        NUM_REL, 8 * D_IN, 8 * D_HID)
    y1p = _tc_prep(xg, wbig1)
    y1 = y1p.reshape(NUM_REL * N_NODES, D_HID)

    zrow16 = jnp.zeros((ZROWS, D_HID), jnp.float32)
    z1d = jnp.zeros((Z1DL,), jnp.float32)
    cnts0, cnts1 = _sc_cnt(dstp, z1d)
    cnts3 = jnp.stack([cnts0, cnts1]).reshape(NC, NPAD, 1)
    (sums1,) = _sc_pass1(idxp, dstp, y1, zrow16)

    w2p = jnp.pad(W2, ((0, 0), (0, 0), (0, D_OUT_PAD - D_OUT)))
    root2p = jnp.pad(root2, ((0, 0), (0, D_OUT_PAD - D_OUT)))
    b2r = jnp.pad(b2, (0, D_OUT_PAD - D_OUT)).reshape(1, D_OUT_PAD)
    eye16 = jnp.eye(16, dtype=jnp.float32)
    wbig2 = jnp.einsum('jk,rfo->rjfko', eye16, w2p).reshape(
        NUM_REL, 16 * D_HID, 16 * D_OUT_PAD)
    h, xr2 = _tc_mid_a(sums1, cnts3, x, root1, b1.reshape(1, D_HID),
                       root2p, b2r)
    hg = h.reshape(NG16, 16 * D_HID)
    y2p = _tc_mid_b(hg, wbig2)
    y2 = y2p.reshape(NUM_REL * N_NODES, D_OUT_PAD)

    zrow8 = jnp.zeros((ZROWS, D_OUT_PAD), jnp.float32)
    (sums2,) = _sc_pass2(idxp, dstp, y2, zrow8)
    return _tc_final(sums2, cnts3, xr2)


# speedup vs baseline: 2.5471x; 1.0421x over previous
import functools

import jax
import jax.numpy as jnp
from jax import lax
from jax.experimental import pallas as pl
from jax.experimental.pallas import tpu as pltpu
from jax.experimental.pallas import tpu_sc as plsc

N_NODES = 100000
N_EDGES = 3200000
NUM_REL = 16
D_IN = 7
D_HID = 16
D_OUT = 2
D_OUT_PAD = 8

NC, NS = 2, 16
NW = NC * NS
EB = 128
EPAD = 3276800
RTOT = EPAD // EB
ROWS_PER_TILE = RTOT // NW
NPAD = N_NODES + 96
RPT_OUT = NPAD // NS

BN = 2000
GRID_N = N_NODES // BN

ZROWS = 512
Z1DL = 8192


def _idx_body(src_ref, et_ref, idx_ref):
    idx_ref[...] = et_ref[...] * N_NODES + src_ref[...]


def _tc_idx(srcp, etp):
    blk = pl.BlockSpec((512, EB), lambda i: (i, 0))
    return pl.pallas_call(
        _idx_body,
        grid=(RTOT // 512,),
        in_specs=[blk, blk],
        out_specs=blk,
        out_shape=jax.ShapeDtypeStruct((RTOT, EB), jnp.int32),
    )(srcp, etp)


NG8 = N_NODES // 8
NG16 = N_NODES // 16


def _prep_body(xg_ref, w_ref, y_ref):
    xg = xg_ref[...]
    ya = jnp.dot(xg, w_ref[0], preferred_element_type=jnp.float32)
    yb = jnp.dot(xg, w_ref[1], preferred_element_type=jnp.float32)
    y_ref[...] = jnp.concatenate([ya, yb], axis=0)


def _tc_prep(xg, wbig1):
    return pl.pallas_call(
        _prep_body,
        grid=(NUM_REL // 2,),
        in_specs=[
            pl.BlockSpec((NG8, 8 * D_IN), lambda p: (0, 0)),
            pl.BlockSpec((2, 8 * D_IN, 128), lambda p: (p, 0, 0)),
        ],
        out_specs=pl.BlockSpec((2 * NG8, 128), lambda p: (p, 0)),
        out_shape=jax.ShapeDtypeStruct((NUM_REL * NG8, 128), jnp.float32),
    )(xg, wbig1)


ROWS_C0 = 1024
ROWS_C1 = 576


def _make_sc_pass(width, kb, async_scatter):
    mesh = plsc.VectorSubcoreMesh(core_axis_name="c", subcore_axis_name="s",
                                  num_cores=NC, num_subcores=NS)

    scratch = [
        pltpu.VMEM((2 * kb, EB), jnp.int32),
        pltpu.VMEM((2 * kb, EB), jnp.int32),
        pltpu.VMEM((2 * kb * EB, width), jnp.float32),
        pltpu.VMEM((ZROWS, width), jnp.float32),
        pltpu.VMEM_SHARED((NPAD, width), jnp.float32),
        pltpu.SemaphoreType.DMA,
        pltpu.SemaphoreType.DMA,
    ]
    if async_scatter:
        scratch += [pltpu.SemaphoreType.DMA, pltpu.SemaphoreType.DMA]
    out_type = [jax.ShapeDtypeStruct((NC, NPAD, width), jnp.float32)]

    @functools.partial(
        pl.kernel, out_type=out_type, mesh=mesh, scratch_types=scratch,
        compiler_params=pltpu.CompilerParams(use_tc_tiling_on_sc=False))
    def sc_pass(idx_hbm, dst_hbm, tab_hbm, zrow_hbm, *refs):
        if async_scatter:
            (sums_hbm, idx_v, dst_v, rows_v, zv, acc_sh,
             gsem0, gsem1, ssem0, ssem1) = refs
            ssem = (ssem0, ssem1)
        else:
            (sums_hbm, idx_v, dst_v, rows_v, zv, acc_sh,
             gsem0, gsem1) = refs
        gsem = (gsem0, gsem1)
        c = lax.axis_index("c")
        s = lax.axis_index("s")
        rbase = jnp.where(c == 0, s * ROWS_C0, NS * ROWS_C0 + s * ROWS_C1)
        outer = jnp.where(c == 0, ROWS_C0 // kb, ROWS_C1 // kb)

        pltpu.sync_copy(zrow_hbm, zv)
        zbase = s * RPT_OUT
        nfull = RPT_OUT // ZROWS
        for k in range(nfull):
            pltpu.sync_copy(zv, acc_sh.at[pl.ds(zbase + k * ZROWS, ZROWS)])
        rem = RPT_OUT - nfull * ZROWS
        pltpu.sync_copy(zv.at[pl.ds(0, rem)],
                        acc_sh.at[pl.ds(zbase + nfull * ZROWS, rem)])

        plsc.subcore_barrier()

        def _load_fire(b, chunk):
            r0 = rbase + chunk * kb
            pltpu.sync_copy(idx_hbm.at[pl.ds(r0, kb)],
                            idx_v.at[pl.ds(b * kb, kb)])
            pltpu.sync_copy(dst_hbm.at[pl.ds(r0, kb)],
                            dst_v.at[pl.ds(b * kb, kb)])
            for j in range(kb):
                pltpu.async_copy(tab_hbm.at[idx_v.at[b * kb + j]],
                                 rows_v.at[pl.ds((b * kb + j) * EB, EB)],
                                 gsem[b])

        def _drain(sem, b):
            nwait = (kb * EB) // ZROWS
            for t in range(nwait):
                pltpu.make_async_copy(
                    zrow_hbm,
                    rows_v.at[pl.ds(b * kb * EB + t * ZROWS, ZROWS)],
                    sem).wait()

        def _wait_g(b):
            _drain(gsem[b], b)

        if async_scatter:
            def _fire_s(b):
                for j in range(kb):
                    pltpu.async_copy(
                        rows_v.at[pl.ds((b * kb + j) * EB, EB)],
                        acc_sh.at[dst_v.at[b * kb + j]], ssem[b], add=True)

            def _wait_s(b):
                _drain(ssem[b], b)

            _load_fire(0, 0)
            _load_fire(1, 1)

            def _pipe(k, carry):
                _wait_g(0)
                _fire_s(0)
                _wait_s(0)
                _load_fire(0, 2 * k + 2)
                _wait_g(1)
                _fire_s(1)
                _wait_s(1)
                _load_fire(1, 2 * k + 3)
                return carry
            lax.fori_loop(0, outer // 2 - 1, _pipe, 0)

            _wait_g(0)
            _fire_s(0)
            _wait_s(0)
            _wait_g(1)
            _fire_s(1)
            _wait_s(1)
        else:
            def _scat_sync(b):
                for j in range(kb):
                    pltpu.sync_copy(rows_v.at[pl.ds((b * kb + j) * EB, EB)],
                                    acc_sh.at[dst_v.at[b * kb + j]],
                                    add=True)

            _load_fire(0, 0)

            def _pipe(k, carry):
                _wait_g(0)
                _load_fire(1, 2 * k + 1)
                _scat_sync(0)
                _wait_g(1)
                _load_fire(0, 2 * k + 2)
                _scat_sync(1)
                return carry
            lax.fori_loop(0, outer // 2 - 1, _pipe, 0)

            _wait_g(0)
            _load_fire(1, outer - 1)
            _scat_sync(0)
            _wait_g(1)
            _scat_sync(1)

        plsc.subcore_barrier()

        ob = s * RPT_OUT
        pltpu.sync_copy(acc_sh.at[pl.ds(ob, RPT_OUT)],
                        sums_hbm.at[c, pl.ds(ob, RPT_OUT)])

    return sc_pass


_sc_pass1 = _make_sc_pass(D_HID, 4, async_scatter=True)
_sc_pass2 = _make_sc_pass(D_OUT_PAD, 8, async_scatter=True)


def _make_sc_cnt(kb):
    mesh = plsc.VectorSubcoreMesh(core_axis_name="c", subcore_axis_name="s",
                                  num_cores=NC, num_subcores=NS)
    outer = ROWS_PER_TILE // kb
    scratch = [
        pltpu.VMEM((kb, EB), jnp.int32),
        pltpu.VMEM((EB,), jnp.float32),
        pltpu.VMEM((Z1DL,), jnp.float32),
        pltpu.VMEM_SHARED((NPAD,), jnp.float32),
    ]
    out_type = [jax.ShapeDtypeStruct((NPAD,), jnp.float32),
                jax.ShapeDtypeStruct((NPAD,), jnp.float32)]

    @functools.partial(
        pl.kernel, out_type=out_type, mesh=mesh, scratch_types=scratch,
        compiler_params=pltpu.CompilerParams(use_tc_tiling_on_sc=False))
    def sc_cnt(dst_hbm, z1d_hbm, cnts0_hbm, cnts1_hbm, dst_v, ones_v, z1_v,
               cnt_sh):
        c = lax.axis_index("c")
        s = lax.axis_index("s")
        wid = c * NS + s

        pltpu.sync_copy(z1d_hbm, z1_v)

        @pl.when(s == 0)
        def _zero_cnt():
            nf1 = NPAD // Z1DL
            for k in range(nf1):
                pltpu.sync_copy(z1_v, cnt_sh.at[pl.ds(k * Z1DL, Z1DL)])
            r1 = NPAD - nf1 * Z1DL
            pltpu.sync_copy(z1_v.at[pl.ds(0, r1)],
                            cnt_sh.at[pl.ds(nf1 * Z1DL, r1)])

        def _init_ones(i, carry):
            ones_v[pl.ds(i * 16, 16)] = jnp.full((16,), 1.0, jnp.float32)
            return carry
        lax.fori_loop(0, EB // 16, _init_ones, 0)

        plsc.subcore_barrier()

        def _step(jo, carry):
            r0 = wid * ROWS_PER_TILE + jo * kb
            pltpu.sync_copy(dst_hbm.at[pl.ds(r0, kb)], dst_v)
            for j in range(kb):
                pltpu.sync_copy(ones_v, cnt_sh.at[dst_v.at[j]], add=True)
            return carry
        lax.fori_loop(0, outer, _step, 0)

        plsc.subcore_barrier()

        @pl.when((s == 0) & (c == 0))
        def _cnt_out0():
            pltpu.sync_copy(cnt_sh, cnts0_hbm)

        @pl.when((s == 0) & (c == 1))
        def _cnt_out1():
            pltpu.sync_copy(cnt_sh, cnts1_hbm)

    return sc_cnt


_sc_cnt = _make_sc_cnt(8)


def _mid_a_body(sums_ref, cnts_ref, x_ref, root1_ref, b1_ref,
                root2_ref, b2_ref, h_ref, xr2_ref):
    agg = sums_ref[0] + sums_ref[1]
    cnt = cnts_ref[0, :, 0] + cnts_ref[1, :, 0]
    inv = 1.0 / jnp.maximum(cnt, 1.0)
    h = agg * inv[:, None] + jnp.dot(
        x_ref[...], root1_ref[...], preferred_element_type=jnp.float32)
    h = jnp.maximum(h + b1_ref[...], 0.0)
    h_ref[...] = h
    xr2_ref[...] = jnp.dot(h, root2_ref[...],
                           preferred_element_type=jnp.float32) + b2_ref[...]


def _tc_mid_a(sums1, cnts, x, root1, b1r, root2p, b2r):
    return pl.pallas_call(
        _mid_a_body,
        grid=(GRID_N,),
        in_specs=[
            pl.BlockSpec((NC, BN, D_HID), lambda i: (0, i, 0)),
            pl.BlockSpec((NC, BN, 1), lambda i: (0, i, 0)),
            pl.BlockSpec((BN, D_IN), lambda i: (i, 0)),
            pl.BlockSpec((D_IN, D_HID), lambda i: (0, 0)),
            pl.BlockSpec((1, D_HID), lambda i: (0, 0)),
            pl.BlockSpec((D_HID, D_OUT_PAD), lambda i: (0, 0)),
            pl.BlockSpec((1, D_OUT_PAD), lambda i: (0, 0)),
        ],
        out_specs=[
            pl.BlockSpec((BN, D_HID), lambda i: (i, 0)),
            pl.BlockSpec((BN, D_OUT_PAD), lambda i: (i, 0)),
        ],
        out_shape=[
            jax.ShapeDtypeStruct((N_NODES, D_HID), jnp.float32),
            jax.ShapeDtypeStruct((N_NODES, D_OUT_PAD), jnp.float32),
        ],
    )(sums1, cnts, x, root1, b1r, root2p, b2r)


def _mid_b_body(hg_ref, w_ref, y_ref):
    hg = hg_ref[...]
    ys = [jnp.dot(hg, w_ref[k], preferred_element_type=jnp.float32)
          for k in range(4)]
    y_ref[...] = jnp.concatenate(ys, axis=0)


def _tc_mid_b(hg, wbig2):
    return pl.pallas_call(
        _mid_b_body,
        grid=(NUM_REL // 4,),
        in_specs=[
            pl.BlockSpec((NG16, 16 * D_HID), lambda q: (0, 0)),
            pl.BlockSpec((4, 16 * D_HID, 128), lambda q: (q, 0, 0)),
        ],
        out_specs=pl.BlockSpec((4 * NG16, 128), lambda q: (q, 0)),
        out_shape=jax.ShapeDtypeStruct((NUM_REL * NG16, 128), jnp.float32),
    )(hg, wbig2)


def _final_body(sums_ref, cnts_ref, xr2_ref, out_ref):
    agg = sums_ref[0] + sums_ref[1]
    cnt = cnts_ref[0, :, 0] + cnts_ref[1, :, 0]
    inv = 1.0 / jnp.maximum(cnt, 1.0)
    z = agg * inv[:, None] + xr2_ref[...]
    lane = lax.broadcasted_iota(jnp.int32, z.shape, 1)
    zm = jnp.where(lane < D_OUT, z, -jnp.inf)
    m = jnp.max(zm, axis=1, keepdims=True)
    lse = m + jnp.log(jnp.sum(jnp.where(lane < D_OUT, jnp.exp(z - m), 0.0),
                              axis=1, keepdims=True))
    out_ref[...] = (z - lse)[:, :D_OUT]


def _tc_final(sums2, cnts, xr2):
    return pl.pallas_call(
        _final_body,
        grid=(GRID_N,),
        in_specs=[
            pl.BlockSpec((NC, BN, D_OUT_PAD), lambda i: (0, i, 0)),
            pl.BlockSpec((NC, BN, 1), lambda i: (0, i, 0)),
            pl.BlockSpec((BN, D_OUT_PAD), lambda i: (i, 0)),
        ],
        out_specs=pl.BlockSpec((BN, D_OUT), lambda i: (i, 0)),
        out_shape=jax.ShapeDtypeStruct((N_NODES, D_OUT), jnp.float32),
    )(sums2, cnts, xr2)


def kernel(x, edge_index, edge_type, W1, root1, b1, W2, root2, b2):
    pad = EPAD - N_EDGES
    srcp = jnp.pad(edge_index[0], (0, pad)).reshape(RTOT, EB)
    etp = jnp.pad(edge_type, (0, pad)).reshape(RTOT, EB)
    dstp = jnp.pad(edge_index[1], (0, pad),
                   constant_values=N_NODES).reshape(RTOT, EB)

    idxp = _tc_idx(srcp, etp)

    xg = x.reshape(NG8, 8 * D_IN)
    eye8 = jnp.eye(8, dtype=jnp.float32)
    wbig1 = jnp.einsum('jk,rfo->rjfko', eye8, W1).reshape(
        NUM_REL, 8 * D_IN, 8 * D_HID)
    y1p = _tc_prep(xg, wbig1)
    y1 = y1p.reshape(NUM_REL * N_NODES, D_HID)

    zrow16 = jnp.zeros((ZROWS, D_HID), jnp.float32)
    z1d = jnp.zeros((Z1DL,), jnp.float32)
    cnts0, cnts1 = _sc_cnt(dstp, z1d)
    cnts3 = jnp.stack([cnts0, cnts1]).reshape(NC, NPAD, 1)
    (sums1,) = _sc_pass1(idxp, dstp, y1, zrow16)

    w2p = jnp.pad(W2, ((0, 0), (0, 0), (0, D_OUT_PAD - D_OUT)))
    root2p = jnp.pad(root2, ((0, 0), (0, D_OUT_PAD - D_OUT)))
    b2r = jnp.pad(b2, (0, D_OUT_PAD - D_OUT)).reshape(1, D_OUT_PAD)
    eye16 = jnp.eye(16, dtype=jnp.float32)
    wbig2 = jnp.einsum('jk,rfo->rjfko', eye16, w2p).reshape(
        NUM_REL, 16 * D_HID, 16 * D_OUT_PAD)
    h, xr2 = _tc_mid_a(sums1, cnts3, x, root1, b1.reshape(1, D_HID),
                       root2p, b2r)
    hg = h.reshape(NG16, 16 * D_HID)
    y2p = _tc_mid_b(hg, wbig2)
    y2 = y2p.reshape(NUM_REL * N_NODES, D_OUT_PAD)

    zrow8 = jnp.zeros((ZROWS, D_OUT_PAD), jnp.float32)
    (sums2,) = _sc_pass2(idxp, dstp, y2, zrow8)
    return _tc_final(sums2, cnts3, xr2)


# speedup vs baseline: 2.9833x; 1.1713x over previous
import functools

import jax
import jax.numpy as jnp
from jax import lax
from jax.experimental import pallas as pl
from jax.experimental.pallas import tpu as pltpu
from jax.experimental.pallas import tpu_sc as plsc

N_NODES = 100000
N_EDGES = 3200000
NUM_REL = 16
D_IN = 7
D_HID = 16
D_OUT = 2
D_OUT_PAD = 8

NC, NS = 2, 16
NW = NC * NS
EB = 128
EPAD = 3276800
RTOT = EPAD // EB
ROWS_PER_TILE = RTOT // NW
NPAD = N_NODES + 96
RPT_OUT = NPAD // NS

BN = 2000
GRID_N = N_NODES // BN

ZROWS = 512
Z1DL = 8192


def _idx_body(src_ref, et_ref, idx_ref):
    idx_ref[...] = et_ref[...] * NPAD + src_ref[...]


def _tc_idx(srcp, etp):
    blk = pl.BlockSpec((512, EB), lambda i: (i, 0))
    return pl.pallas_call(
        _idx_body,
        grid=(RTOT // 512,),
        in_specs=[blk, blk],
        out_specs=blk,
        out_shape=jax.ShapeDtypeStruct((RTOT, EB), jnp.int32),
    )(srcp, etp)


NG8 = NPAD // 8
NG16 = NPAD // 16


def _prep_body(xg_ref, w_ref, y_ref):
    y_ref[...] = jnp.dot(xg_ref[...], w_ref[0],
                         preferred_element_type=jnp.float32,
                 precision=lax.Precision.HIGHEST)


def _tc_prep(xg, wbig1):
    return pl.pallas_call(
        _prep_body,
        grid=(NUM_REL,),
        in_specs=[
            pl.BlockSpec((NG8, 8 * D_IN), lambda r: (0, 0)),
            pl.BlockSpec((1, 8 * D_IN, 128), lambda r: (r, 0, 0)),
        ],
        out_specs=pl.BlockSpec((NG8, 128), lambda r: (r, 0)),
        out_shape=jax.ShapeDtypeStruct((NUM_REL * NG8, 128), jnp.float32),
    )(xg, wbig1)


ROWS_C0 = 1024
ROWS_C1 = 576


def _make_sc_pass(width, kb, async_scatter):
    mesh = plsc.VectorSubcoreMesh(core_axis_name="c", subcore_axis_name="s",
                                  num_cores=NC, num_subcores=NS)

    scratch = [
        pltpu.VMEM((2 * kb, EB), jnp.int32),
        pltpu.VMEM((2 * kb, EB), jnp.int32),
        pltpu.VMEM((2 * kb * EB, width), jnp.float32),
        pltpu.VMEM((ZROWS, width), jnp.float32),
        pltpu.VMEM_SHARED((NPAD, width), jnp.float32),
        pltpu.SemaphoreType.DMA,
        pltpu.SemaphoreType.DMA,
    ]
    if async_scatter:
        scratch += [pltpu.SemaphoreType.DMA, pltpu.SemaphoreType.DMA]
    out_type = [jax.ShapeDtypeStruct((NC, NPAD, width), jnp.float32)]

    @functools.partial(
        pl.kernel, out_type=out_type, mesh=mesh, scratch_types=scratch,
        compiler_params=pltpu.CompilerParams(use_tc_tiling_on_sc=False))
    def sc_pass(idx_hbm, dst_hbm, tab_hbm, zrow_hbm, *refs):
        if async_scatter:
            (sums_hbm, idx_v, dst_v, rows_v, zv, acc_sh,
             gsem0, gsem1, ssem0, ssem1) = refs
            ssem = (ssem0, ssem1)
        else:
            (sums_hbm, idx_v, dst_v, rows_v, zv, acc_sh,
             gsem0, gsem1) = refs
        gsem = (gsem0, gsem1)
        c = lax.axis_index("c")
        s = lax.axis_index("s")
        rbase = jnp.where(c == 0, s * ROWS_C0, NS * ROWS_C0 + s * ROWS_C1)
        outer = jnp.where(c == 0, ROWS_C0 // kb, ROWS_C1 // kb)

        pltpu.sync_copy(zrow_hbm, zv)
        zbase = s * RPT_OUT
        nfull = RPT_OUT // ZROWS
        for k in range(nfull):
            pltpu.sync_copy(zv, acc_sh.at[pl.ds(zbase + k * ZROWS, ZROWS)])
        rem = RPT_OUT - nfull * ZROWS
        pltpu.sync_copy(zv.at[pl.ds(0, rem)],
                        acc_sh.at[pl.ds(zbase + nfull * ZROWS, rem)])

        plsc.subcore_barrier()

        def _load_fire(b, chunk):
            r0 = rbase + chunk * kb
            pltpu.sync_copy(idx_hbm.at[pl.ds(r0, kb)],
                            idx_v.at[pl.ds(b * kb, kb)])
            pltpu.sync_copy(dst_hbm.at[pl.ds(r0, kb)],
                            dst_v.at[pl.ds(b * kb, kb)])
            for j in range(kb):
                pltpu.async_copy(tab_hbm.at[idx_v.at[b * kb + j]],
                                 rows_v.at[pl.ds((b * kb + j) * EB, EB)],
                                 gsem[b])

        def _drain(sem, b):
            nwait = (kb * EB) // ZROWS
            for t in range(nwait):
                pltpu.make_async_copy(
                    zrow_hbm,
                    rows_v.at[pl.ds(b * kb * EB + t * ZROWS, ZROWS)],
                    sem).wait()

        def _wait_g(b):
            _drain(gsem[b], b)

        if async_scatter:
            def _fire_s(b):
                for j in range(kb):
                    pltpu.async_copy(
                        rows_v.at[pl.ds((b * kb + j) * EB, EB)],
                        acc_sh.at[dst_v.at[b * kb + j]], ssem[b], add=True)

            def _wait_s(b):
                _drain(ssem[b], b)

            _load_fire(0, 0)
            _load_fire(1, 1)

            def _pipe(k, carry):
                _wait_g(0)
                _fire_s(0)
                _wait_s(0)
                _load_fire(0, 2 * k + 2)
                _wait_g(1)
                _fire_s(1)
                _wait_s(1)
                _load_fire(1, 2 * k + 3)
                return carry
            lax.fori_loop(0, outer // 2 - 1, _pipe, 0)

            _wait_g(0)
            _fire_s(0)
            _wait_s(0)
            _wait_g(1)
            _fire_s(1)
            _wait_s(1)
        else:
            def _scat_sync(b):
                for j in range(kb):
                    pltpu.sync_copy(rows_v.at[pl.ds((b * kb + j) * EB, EB)],
                                    acc_sh.at[dst_v.at[b * kb + j]],
                                    add=True)

            _load_fire(0, 0)

            def _pipe(k, carry):
                _wait_g(0)
                _load_fire(1, 2 * k + 1)
                _scat_sync(0)
                _wait_g(1)
                _load_fire(0, 2 * k + 2)
                _scat_sync(1)
                return carry
            lax.fori_loop(0, outer // 2 - 1, _pipe, 0)

            _wait_g(0)
            _load_fire(1, outer - 1)
            _scat_sync(0)
            _wait_g(1)
            _scat_sync(1)

        plsc.subcore_barrier()

        ob = s * RPT_OUT
        pltpu.sync_copy(acc_sh.at[pl.ds(ob, RPT_OUT)],
                        sums_hbm.at[c, pl.ds(ob, RPT_OUT)])

    return sc_pass


_sc_pass1 = _make_sc_pass(D_HID, 4, async_scatter=True)
_sc_pass2 = _make_sc_pass(D_OUT_PAD, 8, async_scatter=True)


def _make_sc_cnt(kb):
    mesh = plsc.VectorSubcoreMesh(core_axis_name="c", subcore_axis_name="s",
                                  num_cores=NC, num_subcores=NS)
    outer = ROWS_PER_TILE // kb
    scratch = [
        pltpu.VMEM((kb, EB), jnp.int32),
        pltpu.VMEM((EB,), jnp.float32),
        pltpu.VMEM((Z1DL,), jnp.float32),
        pltpu.VMEM_SHARED((NPAD,), jnp.float32),
    ]
    out_type = [jax.ShapeDtypeStruct((NPAD,), jnp.float32),
                jax.ShapeDtypeStruct((NPAD,), jnp.float32)]

    @functools.partial(
        pl.kernel, out_type=out_type, mesh=mesh, scratch_types=scratch,
        compiler_params=pltpu.CompilerParams(use_tc_tiling_on_sc=False))
    def sc_cnt(dst_hbm, z1d_hbm, cnts0_hbm, cnts1_hbm, dst_v, ones_v, z1_v,
               cnt_sh):
        c = lax.axis_index("c")
        s = lax.axis_index("s")
        wid = c * NS + s

        pltpu.sync_copy(z1d_hbm, z1_v)

        @pl.when(s == 0)
        def _zero_cnt():
            nf1 = NPAD // Z1DL
            for k in range(nf1):
                pltpu.sync_copy(z1_v, cnt_sh.at[pl.ds(k * Z1DL, Z1DL)])
            r1 = NPAD - nf1 * Z1DL
            pltpu.sync_copy(z1_v.at[pl.ds(0, r1)],
                            cnt_sh.at[pl.ds(nf1 * Z1DL, r1)])

        def _init_ones(i, carry):
            ones_v[pl.ds(i * 16, 16)] = jnp.full((16,), 1.0, jnp.float32)
            return carry
        lax.fori_loop(0, EB // 16, _init_ones, 0)

        plsc.subcore_barrier()

        def _step(jo, carry):
            r0 = wid * ROWS_PER_TILE + jo * kb
            pltpu.sync_copy(dst_hbm.at[pl.ds(r0, kb)], dst_v)
            for j in range(kb):
                pltpu.sync_copy(ones_v, cnt_sh.at[dst_v.at[j]], add=True)
            return carry
        lax.fori_loop(0, outer, _step, 0)

        plsc.subcore_barrier()

        @pl.when((s == 0) & (c == 0))
        def _cnt_out0():
            pltpu.sync_copy(cnt_sh, cnts0_hbm)

        @pl.when((s == 0) & (c == 1))
        def _cnt_out1():
            pltpu.sync_copy(cnt_sh, cnts1_hbm)

    return sc_cnt


_sc_cnt = _make_sc_cnt(8)


def _mid_a_body(sums_ref, cnts_ref, xg_ref, rtbd_ref, b1t_ref, exp8_ref,
                h_ref):
    sp = sums_ref[...]
    agg = sp[0] + sp[1]
    c8 = cnts_ref[0] + cnts_ref[1]
    inv8 = 1.0 / jnp.maximum(c8, 1.0)
    inv128 = jnp.dot(inv8, exp8_ref[...],
                     preferred_element_type=jnp.float32,
                 precision=lax.Precision.HIGHEST)
    rt = jnp.dot(xg_ref[...], rtbd_ref[...],
                 preferred_element_type=jnp.float32,
                 precision=lax.Precision.HIGHEST)
    h = agg * inv128 + rt + b1t_ref[...]
    h_ref[...] = jnp.maximum(h, 0.0)


def _tc_mid_a(sums1p, cnts8, xg, rtbd1, b1t, exp8):
    g = NG8 // 2
    return pl.pallas_call(
        _mid_a_body,
        grid=(2,),
        in_specs=[
            pl.BlockSpec((NC, g, 128), lambda i: (0, i, 0)),
            pl.BlockSpec((NC, g, 8), lambda i: (0, i, 0)),
            pl.BlockSpec((g, 8 * D_IN), lambda i: (i, 0)),
            pl.BlockSpec((8 * D_IN, 128), lambda i: (0, 0)),
            pl.BlockSpec((1, 128), lambda i: (0, 0)),
            pl.BlockSpec((8, 128), lambda i: (0, 0)),
        ],
        out_specs=pl.BlockSpec((g, 128), lambda i: (i, 0)),
        out_shape=jax.ShapeDtypeStruct((NG8, 128), jnp.float32),
    )(sums1p, cnts8, xg, rtbd1, b1t, exp8)


def _mid_b_body(hg_ref, w_ref, b2t_ref, y_ref):
    r = pl.program_id(0)
    y = jnp.dot(hg_ref[...], w_ref[0], preferred_element_type=jnp.float32,
                 precision=lax.Precision.HIGHEST)
    y_ref[...] = y + jnp.where(r == NUM_REL, 1.0, 0.0) * b2t_ref[...]


def _tc_mid_b(hg, wbig2, b2t):
    return pl.pallas_call(
        _mid_b_body,
        grid=(NUM_REL + 1,),
        in_specs=[
            pl.BlockSpec((NG16, 16 * D_HID), lambda r: (0, 0)),
            pl.BlockSpec((1, 16 * D_HID, 128), lambda r: (r, 0, 0)),
            pl.BlockSpec((1, 128), lambda r: (0, 0)),
        ],
        out_specs=pl.BlockSpec((NG16, 128), lambda r: (r, 0)),
        out_shape=jax.ShapeDtypeStruct(((NUM_REL + 1) * NG16, 128),
                                       jnp.float32),
    )(hg, wbig2, b2t)


def _final_body(sums_ref, cnts_ref, xr2_ref, exp16_ref, sel0_ref, sel1_ref,
                o0_ref, o1_ref):
    sp = sums_ref[...]
    agg = sp[0] + sp[1]
    c16 = cnts_ref[0] + cnts_ref[1]
    inv16 = 1.0 / jnp.maximum(c16, 1.0)
    inv128 = jnp.dot(inv16, exp16_ref[...],
                     preferred_element_type=jnp.float32,
                 precision=lax.Precision.HIGHEST)
    z = agg * inv128 + xr2_ref[...]
    z0 = jnp.dot(z, sel0_ref[...], preferred_element_type=jnp.float32,
                 precision=lax.Precision.HIGHEST)
    z1 = jnp.dot(z, sel1_ref[...], preferred_element_type=jnp.float32,
                 precision=lax.Precision.HIGHEST)
    m = jnp.maximum(z0, z1)
    lse = m + jnp.log(jnp.exp(z0 - m) + jnp.exp(z1 - m))
    o0_ref[...] = z0 - lse
    o1_ref[...] = z1 - lse


def _tc_final(sums2p, cnts16, xr2p, exp16, sel0, sel1):
    g = NG16 // 2
    return pl.pallas_call(
        _final_body,
        grid=(2,),
        in_specs=[
            pl.BlockSpec((NC, g, 128), lambda i: (0, i, 0)),
            pl.BlockSpec((NC, g, 16), lambda i: (0, i, 0)),
            pl.BlockSpec((g, 128), lambda i: (i, 0)),
            pl.BlockSpec((16, 128), lambda i: (0, 0)),
            pl.BlockSpec((128, 16), lambda i: (0, 0)),
            pl.BlockSpec((128, 16), lambda i: (0, 0)),
        ],
        out_specs=[
            pl.BlockSpec((g, 16), lambda i: (i, 0)),
            pl.BlockSpec((g, 16), lambda i: (i, 0)),
        ],
        out_shape=[
            jax.ShapeDtypeStruct((NG16, 16), jnp.float32),
            jax.ShapeDtypeStruct((NG16, 16), jnp.float32),
        ],
    )(sums2p, cnts16, xr2p, exp16, sel0, sel1)


def kernel(x, edge_index, edge_type, W1, root1, b1, W2, root2, b2):
    pad = EPAD - N_EDGES
    srcp = jnp.pad(edge_index[0], (0, pad)).reshape(RTOT, EB)
    etp = jnp.pad(edge_type, (0, pad)).reshape(RTOT, EB)
    dstp = jnp.pad(edge_index[1], (0, pad),
                   constant_values=N_NODES).reshape(RTOT, EB)

    idxp = _tc_idx(srcp, etp)

    xp = jnp.pad(x, ((0, NPAD - N_NODES), (0, 0)))
    xg = xp.reshape(NG8, 8 * D_IN)
    eye8 = jnp.eye(8, dtype=jnp.float32)
    eye16 = jnp.eye(16, dtype=jnp.float32)
    wbig1 = jnp.einsum('jk,rfo->rjfko', eye8, W1).reshape(
        NUM_REL, 8 * D_IN, 8 * D_HID)
    y1p = _tc_prep(xg, wbig1)
    y1 = y1p.reshape(NUM_REL * NPAD, D_HID)

    zrow16 = jnp.zeros((ZROWS, D_HID), jnp.float32)
    z1d = jnp.zeros((Z1DL,), jnp.float32)
    cnts0, cnts1 = _sc_cnt(dstp, z1d)
    cnts = jnp.stack([cnts0, cnts1])
    cnts8 = cnts.reshape(NC, NG8, 8)
    cnts16 = cnts.reshape(NC, NG16, 16)
    (sums1,) = _sc_pass1(idxp, dstp, y1, zrow16)
    sums1p = sums1.reshape(NC, NG8, 128)

    rtbd1 = jnp.einsum('jk,fo->jfko', eye8, root1).reshape(8 * D_IN, 128)
    b1t = jnp.tile(b1, 8).reshape(1, 128)
    exp8 = jnp.repeat(eye8, 16, axis=1)
    h8 = _tc_mid_a(sums1p, cnts8, xg, rtbd1, b1t, exp8)
    hg = h8.reshape(NG16, 16 * D_HID)

    w2p = jnp.pad(W2, ((0, 0), (0, 0), (0, D_OUT_PAD - D_OUT)))
    root2p = jnp.pad(root2, ((0, 0), (0, D_OUT_PAD - D_OUT)))
    b2p = jnp.pad(b2, (0, D_OUT_PAD - D_OUT))
    wbig2 = jnp.einsum('jk,rfo->rjfko', eye16, w2p).reshape(
        NUM_REL, 16 * D_HID, 128)
    rtbd2 = jnp.einsum('jk,fo->jfko', eye16, root2p).reshape(
        1, 16 * D_HID, 128)
    wbig2ext = jnp.concatenate([wbig2, rtbd2], axis=0)
    b2t = jnp.tile(b2p, 16).reshape(1, 128)
    y2ext = _tc_mid_b(hg, wbig2ext, b2t)
    y2 = y2ext.reshape((NUM_REL + 1) * NPAD, D_OUT_PAD)
    xr2p = y2ext[NUM_REL * NG16:]

    zrow8 = jnp.zeros((ZROWS, D_OUT_PAD), jnp.float32)
    (sums2,) = _sc_pass2(idxp, dstp, y2, zrow8)
    sums2p = sums2.reshape(NC, NG16, 128)

    exp16 = jnp.repeat(eye16, 8, axis=1)
    lanes = jnp.arange(16)
    sel0 = jnp.zeros((128, 16), jnp.float32).at[lanes * 8, lanes].set(1.0)
    sel1 = jnp.zeros((128, 16), jnp.float32).at[lanes * 8 + 1,
                                                lanes].set(1.0)
    o0, o1 = _tc_final(sums2p, cnts16, xr2p, exp16, sel0, sel1)
    out = jnp.stack([o0.reshape(NPAD), o1.reshape(NPAD)], axis=1)
    return out[:N_NODES]
